# baseline (speedup 1.0000x reference)
import functools

import jax
import jax.numpy as jnp
from jax import lax
from jax.experimental import pallas as pl
from jax.experimental.pallas import tpu as pltpu
from jax.experimental.pallas import tpu_sc as plsc

B = 1024
T = 200
OBS = 512
BB = 32

_INFO = plsc.get_sparse_core_info()
_NC = _INFO.num_cores
_NW = _INFO.num_cores * _INFO.num_subcores
RPW = B // _NW


def _obs_body(step_ref, obs_blk, buf_blk, out_blk):
    step = step_ref[0]
    tids = jax.lax.broadcasted_iota(jnp.int32, (1, T, 1), 1)
    out_blk[...] = jnp.where(tids == step, obs_blk[...][:, None, :], buf_blk[...])


def _sc_body(step_hbm, act_hbm, rew_hbm, logp_hbm, val_hbm,
             abuf_hbm, rbuf_hbm, lbuf_hbm, vbuf_hbm,
             aout_hbm, rout_hbm, lout_hbm, vout_hbm,
             step_v, fvals_v, ivals_v, fbuf_v, ibuf_v, vbuf_v):
    wid = lax.axis_index("s") * _NC + lax.axis_index("c")
    base = wid * RPW
    pltpu.sync_copy(step_hbm, step_v)
    cols16 = step_v[pl.ds(0, 16)]
    step = cols16[0]
    iota16 = jax.lax.broadcasted_iota(jnp.int32, (16,), 0)
    off16 = pl.multiple_of((step // 16) * 16, 16)
    posv = cols16 - jnp.full((16,), off16, dtype=jnp.int32)
    hitmask = iota16 == posv

    def copy_buf(src_hbm, vals_hbm, dst_hbm, buf_ref, vals_ref, ncol):
        pltpu.sync_copy(src_hbm.at[pl.ds(base, RPW)], buf_ref)
        pltpu.sync_copy(vals_hbm.at[pl.ds(base, RPW)], vals_ref)
        for i in range(RPW):
            vc = vals_ref[pl.ds((i // 16) * 16, 16)]
            vi = vc[i % 16]
            row = buf_ref.at[i]
            chunk = row[pl.ds(off16, 16)]
            row[pl.ds(off16, 16)] = jnp.where(
                hitmask, jnp.full((16,), vi, dtype=vals_ref.dtype), chunk)
        pltpu.sync_copy(buf_ref, dst_hbm.at[pl.ds(base, RPW)])

    copy_buf(abuf_hbm, act_hbm, aout_hbm, ibuf_v, ivals_v, T)
    copy_buf(rbuf_hbm, rew_hbm, rout_hbm, fbuf_v, fvals_v, T)
    copy_buf(lbuf_hbm, logp_hbm, lout_hbm, fbuf_v, fvals_v, T)
    copy_buf(vbuf_hbm, val_hbm, vout_hbm, vbuf_v, fvals_v, T + 1)


_sc_kernel = functools.partial(
    pl.kernel,
    out_type=(
        jax.ShapeDtypeStruct((B, T), jnp.int32),
        jax.ShapeDtypeStruct((B, T), jnp.float32),
        jax.ShapeDtypeStruct((B, T), jnp.float32),
        jax.ShapeDtypeStruct((B, T + 1), jnp.float32),
    ),
    mesh=plsc.VectorSubcoreMesh(core_axis_name="c", subcore_axis_name="s"),
    scratch_types=[
        pltpu.VMEM((16,), jnp.int32),
        pltpu.VMEM((RPW,), jnp.float32),
        pltpu.VMEM((RPW,), jnp.int32),
        pltpu.VMEM((RPW, T), jnp.float32),
        pltpu.VMEM((RPW, T), jnp.int32),
        pltpu.VMEM((RPW, T + 1), jnp.float32),
    ],
)(_sc_body)


def kernel(step, obs, action, reward, log_prob, value,
           obs_buf, actions_buf, rewards_buf, log_prob_buf, values_buf):
    step_i32 = jnp.asarray(step, dtype=jnp.int32)
    step_arr = step_i32.reshape((1,))

    new_obs = pl.pallas_call(
        _obs_body,
        grid_spec=pltpu.PrefetchScalarGridSpec(
            num_scalar_prefetch=1,
            grid=(B // BB,),
            in_specs=[
                pl.BlockSpec((BB, OBS), lambda i, s: (i, 0)),
                pl.BlockSpec((BB, T, OBS), lambda i, s: (i, 0, 0)),
            ],
            out_specs=pl.BlockSpec((BB, T, OBS), lambda i, s: (i, 0, 0)),
        ),
        out_shape=jax.ShapeDtypeStruct((B, T, OBS), jnp.float32),
        compiler_params=pltpu.CompilerParams(
            dimension_semantics=("arbitrary",),
        ),
    )(step_arr, obs, obs_buf)

    step_vec = jnp.full((16,), step_i32, dtype=jnp.int32)
    new_actions, new_rewards, new_log_prob, new_values = _sc_kernel(
        step_vec, action, reward, log_prob, value,
        actions_buf, rewards_buf, log_prob_buf, values_buf)

    return (new_obs, new_actions, new_rewards, new_log_prob, new_values)
